# baseline (speedup 1.0000x reference)
import jax
import jax.numpy as jnp
from jax.experimental import pallas as pl
from jax.experimental.pallas import tpu as pltpu
from jax.experimental.pallas import tpu_sc as plsc

N = 10000
DEG = 32
D_MSG = 128
D_FEAT = 128
D_HID = 128
H = 256
OUT = 128

BN = 400
NS = 2400
SBN = 8


def _sc_sum_body(in_vmem, out_vmem):
    @pl.loop(0, SBN)
    def _node(r):
        base = r * (DEG * D_MSG)

        @pl.loop(0, D_MSG, step=16)
        def _lane(c):
            acc = in_vmem[pl.ds(base + c, 16)]
            for d in range(1, DEG):
                acc = acc + in_vmem[pl.ds(base + d * D_MSG + c, 16)]
            out_vmem[pl.ds(r * D_MSG + c, 16)] = acc


def _sc_msum(mb_flat):
    @pl.kernel(
        out_type=jax.ShapeDtypeStruct((NS * D_MSG,), jnp.float32),
        mesh=plsc.VectorSubcoreMesh(core_axis_name="c", subcore_axis_name="s"),
    )
    def k(mb_hbm, out_hbm):
        pltpu.emit_pipeline(
            _sc_sum_body,
            grid=(NS // SBN,),
            in_specs=[pl.BlockSpec((SBN * DEG * D_MSG,), lambda i: (i,))],
            out_specs=[pl.BlockSpec((SBN * D_MSG,), lambda i: (i,))],
            core_axis_name=("c", "s"),
            dimension_semantics=(pltpu.PARALLEL,),
        )(mb_hbm, out_hbm)

    return k(mb_flat)


def _mlp(msum, nf_ref, nh_ref, w1m_ref, w1f_ref, w1h_ref, b1_ref,
         w2_ref, b2_ref, w3_ref, b3_ref, o_ref):
    h = (jnp.dot(msum, w1m_ref[...], preferred_element_type=jnp.float32)
         + jnp.dot(nf_ref[...], w1f_ref[...], preferred_element_type=jnp.float32)
         + jnp.dot(nh_ref[...], w1h_ref[...], preferred_element_type=jnp.float32)
         + b1_ref[...])
    h = jnp.maximum(h, 0.0)
    h = jnp.dot(h, w2_ref[...], preferred_element_type=jnp.float32) + b2_ref[...]
    h = jnp.maximum(h, 0.0)
    o_ref[...] = jnp.dot(h, w3_ref[...], preferred_element_type=jnp.float32) + b3_ref[...]


def _fused_body(mb_ref, nf_ref, nh_ref, *rest):
    msum = jnp.sum(mb_ref[...], axis=1)
    _mlp(msum, nf_ref, nh_ref, *rest)


def _msum_body(ms_ref, nf_ref, nh_ref, *rest):
    _mlp(ms_ref[...], nf_ref, nh_ref, *rest)


@jax.jit
def kernel(mailbox, node_features, node_hidden_rep, W1, b1, W2, b2, W3, b3):
    w1m = W1[:D_MSG]
    w1f = W1[D_MSG:D_MSG + D_FEAT]
    w1h = W1[D_MSG + D_FEAT:]
    b1r = b1.reshape(1, H)
    b2r = b2.reshape(1, H)
    b3r = b3.reshape(1, OUT)
    weight_args = (w1m, w1f, w1h, b1r, W2, b2r, W3, b3r)
    weight_specs = [pl.BlockSpec(w.shape, lambda i: (0, 0)) for w in weight_args]

    msum_sc = _sc_msum(mailbox.reshape(-1)).reshape(NS, D_MSG)

    off = NS // BN
    out_hi = pl.pallas_call(
        _fused_body,
        grid=((N - NS) // BN,),
        in_specs=[
            pl.BlockSpec((BN, DEG, D_MSG), lambda i: (i + off, 0, 0)),
            pl.BlockSpec((BN, D_FEAT), lambda i: (i + off, 0)),
            pl.BlockSpec((BN, D_HID), lambda i: (i + off, 0)),
            *weight_specs,
        ],
        out_specs=pl.BlockSpec((BN, OUT), lambda i: (i, 0)),
        out_shape=jax.ShapeDtypeStruct((N - NS, OUT), jnp.float32),
        compiler_params=pltpu.CompilerParams(
            dimension_semantics=("parallel",),
        ),
    )(mailbox, node_features, node_hidden_rep, *weight_args)

    out_lo = pl.pallas_call(
        _msum_body,
        grid=(NS // BN,),
        in_specs=[
            pl.BlockSpec((BN, D_MSG), lambda i: (i, 0)),
            pl.BlockSpec((BN, D_FEAT), lambda i: (i, 0)),
            pl.BlockSpec((BN, D_HID), lambda i: (i, 0)),
            *weight_specs,
        ],
        out_specs=pl.BlockSpec((BN, OUT), lambda i: (i, 0)),
        out_shape=jax.ShapeDtypeStruct((NS, OUT), jnp.float32),
        compiler_params=pltpu.CompilerParams(
            dimension_semantics=("parallel",),
        ),
    )(msum_sc, node_features, node_hidden_rep, *weight_args)

    return jnp.concatenate([out_lo, out_hi], axis=0)


# speedup vs baseline: 1.2018x; 1.2018x over previous
import jax
import jax.numpy as jnp
from jax.experimental import pallas as pl
from jax.experimental.pallas import tpu as pltpu

N = 10000
DEG = 32
D_MSG = 128
D_FEAT = 128
D_HID = 128
H = 256
OUT = 128

BN = 400
NSPLIT = 4


def _fused_body(mb0_ref, mb1_ref, mb2_ref, mb3_ref, nf_ref, nh_ref,
                w1m_ref, w1f_ref, w1h_ref, b1_ref,
                w2_ref, b2_ref, w3_ref, b3_ref, o_ref):
    msum = (jnp.sum(mb0_ref[...], axis=1) + jnp.sum(mb1_ref[...], axis=1)
            + jnp.sum(mb2_ref[...], axis=1) + jnp.sum(mb3_ref[...], axis=1))
    h = (jnp.dot(msum, w1m_ref[...], preferred_element_type=jnp.float32)
         + jnp.dot(nf_ref[...], w1f_ref[...], preferred_element_type=jnp.float32)
         + jnp.dot(nh_ref[...], w1h_ref[...], preferred_element_type=jnp.float32)
         + b1_ref[...])
    h = jnp.maximum(h, 0.0)
    h = jnp.dot(h, w2_ref[...], preferred_element_type=jnp.float32) + b2_ref[...]
    h = jnp.maximum(h, 0.0)
    o_ref[...] = jnp.dot(h, w3_ref[...], preferred_element_type=jnp.float32) + b3_ref[...]


@jax.jit
def kernel(mailbox, node_features, node_hidden_rep, W1, b1, W2, b2, W3, b3):
    w1m = W1[:D_MSG]
    w1f = W1[D_MSG:D_MSG + D_FEAT]
    w1h = W1[D_MSG + D_FEAT:]
    b1r = b1.reshape(1, H)
    b2r = b2.reshape(1, H)
    b3r = b3.reshape(1, OUT)

    grid = (N // BN,)
    dq = DEG // NSPLIT
    mb_specs = [
        pl.BlockSpec((BN, dq, D_MSG), lambda i, q=q: (i, q, 0))
        for q in range(NSPLIT)
    ]

    return pl.pallas_call(
        _fused_body,
        grid=grid,
        in_specs=[
            *mb_specs,
            pl.BlockSpec((BN, D_FEAT), lambda i: (i, 0)),
            pl.BlockSpec((BN, D_HID), lambda i: (i, 0)),
            pl.BlockSpec(w1m.shape, lambda i: (0, 0)),
            pl.BlockSpec(w1f.shape, lambda i: (0, 0)),
            pl.BlockSpec(w1h.shape, lambda i: (0, 0)),
            pl.BlockSpec(b1r.shape, lambda i: (0, 0)),
            pl.BlockSpec(W2.shape, lambda i: (0, 0)),
            pl.BlockSpec(b2r.shape, lambda i: (0, 0)),
            pl.BlockSpec(W3.shape, lambda i: (0, 0)),
            pl.BlockSpec(b3r.shape, lambda i: (0, 0)),
        ],
        out_specs=pl.BlockSpec((BN, OUT), lambda i: (i, 0)),
        out_shape=jax.ShapeDtypeStruct((N, OUT), jnp.float32),
        compiler_params=pltpu.CompilerParams(
            dimension_semantics=("parallel",),
        ),
    )(mailbox, mailbox, mailbox, mailbox, node_features, node_hidden_rep,
      w1m, w1f, w1h, b1r, W2, b2r, W3, b3r)


# speedup vs baseline: 1.4364x; 1.1952x over previous
import jax
import jax.numpy as jnp
from jax.experimental import pallas as pl
from jax.experimental.pallas import tpu as pltpu

N = 10000
DEG = 32
D_MSG = 128
D_FEAT = 128
D_HID = 128
H = 256
OUT = 128

BN = 400
BH = BN // 2


def _fused_body(mb0_ref, mb1_ref, nf_ref, nh_ref,
                w1m_ref, w1f_ref, w1h_ref, b1_ref,
                w2_ref, b2_ref, w3_ref, b3_ref, o_ref):
    msum = jnp.concatenate(
        [jnp.sum(mb0_ref[...], axis=1), jnp.sum(mb1_ref[...], axis=1)], axis=0)
    h = (jnp.dot(msum, w1m_ref[...], preferred_element_type=jnp.float32)
         + jnp.dot(nf_ref[...], w1f_ref[...], preferred_element_type=jnp.float32)
         + jnp.dot(nh_ref[...], w1h_ref[...], preferred_element_type=jnp.float32)
         + b1_ref[...])
    h = jnp.maximum(h, 0.0)
    h = jnp.dot(h, w2_ref[...], preferred_element_type=jnp.float32) + b2_ref[...]
    h = jnp.maximum(h, 0.0)
    o_ref[...] = jnp.dot(h, w3_ref[...], preferred_element_type=jnp.float32) + b3_ref[...]


@jax.jit
def kernel(mailbox, node_features, node_hidden_rep, W1, b1, W2, b2, W3, b3):
    w1m = W1[:D_MSG]
    w1f = W1[D_MSG:D_MSG + D_FEAT]
    w1h = W1[D_MSG + D_FEAT:]
    b1r = b1.reshape(1, H)
    b2r = b2.reshape(1, H)
    b3r = b3.reshape(1, OUT)

    grid = (N // BN,)

    return pl.pallas_call(
        _fused_body,
        grid=grid,
        in_specs=[
            pl.BlockSpec((BH, DEG, D_MSG), lambda i: (2 * i, 0, 0)),
            pl.BlockSpec((BH, DEG, D_MSG), lambda i: (2 * i + 1, 0, 0)),
            pl.BlockSpec((BN, D_FEAT), lambda i: (i, 0)),
            pl.BlockSpec((BN, D_HID), lambda i: (i, 0)),
            pl.BlockSpec(w1m.shape, lambda i: (0, 0)),
            pl.BlockSpec(w1f.shape, lambda i: (0, 0)),
            pl.BlockSpec(w1h.shape, lambda i: (0, 0)),
            pl.BlockSpec(b1r.shape, lambda i: (0, 0)),
            pl.BlockSpec(W2.shape, lambda i: (0, 0)),
            pl.BlockSpec(b2r.shape, lambda i: (0, 0)),
            pl.BlockSpec(W3.shape, lambda i: (0, 0)),
            pl.BlockSpec(b3r.shape, lambda i: (0, 0)),
        ],
        out_specs=pl.BlockSpec((BN, OUT), lambda i: (i, 0)),
        out_shape=jax.ShapeDtypeStruct((N, OUT), jnp.float32),
        compiler_params=pltpu.CompilerParams(
            dimension_semantics=("parallel",),
        ),
    )(mailbox, mailbox, node_features, node_hidden_rep,
      w1m, w1f, w1h, b1r, W2, b2r, W3, b3r)


# speedup vs baseline: 1.4562x; 1.0138x over previous
import jax
import jax.numpy as jnp
from jax.experimental import pallas as pl
from jax.experimental.pallas import tpu as pltpu

N = 10000
DEG = 32
D_MSG = 128
D_FEAT = 128
D_HID = 128
H = 256
OUT = 128

BN = 400


def _fused_body(mb_ref, nf_ref, nh_ref, w1m_ref, w1f_ref, w1h_ref, b1_ref,
                w2_ref, b2_ref, w3_ref, b3_ref, o_ref):
    msum = jnp.sum(mb_ref[...], axis=1)
    h = (jnp.dot(msum, w1m_ref[...], preferred_element_type=jnp.float32)
         + jnp.dot(nf_ref[...], w1f_ref[...], preferred_element_type=jnp.float32)
         + jnp.dot(nh_ref[...], w1h_ref[...], preferred_element_type=jnp.float32)
         + b1_ref[...])
    h = jnp.maximum(h, 0.0)
    h = jnp.dot(h, w2_ref[...], preferred_element_type=jnp.float32) + b2_ref[...]
    h = jnp.maximum(h, 0.0)
    o_ref[...] = jnp.dot(h, w3_ref[...], preferred_element_type=jnp.float32) + b3_ref[...]


@jax.jit
def kernel(mailbox, node_features, node_hidden_rep, W1, b1, W2, b2, W3, b3):
    w1m = W1[:D_MSG]
    w1f = W1[D_MSG:D_MSG + D_FEAT]
    w1h = W1[D_MSG + D_FEAT:]
    b1r = b1.reshape(1, H)
    b2r = b2.reshape(1, H)
    b3r = b3.reshape(1, OUT)

    grid = (N // BN,)

    return pl.pallas_call(
        _fused_body,
        grid=grid,
        in_specs=[
            pl.BlockSpec((BN, DEG, D_MSG), lambda i: (i, 0, 0)),
            pl.BlockSpec((BN, D_FEAT), lambda i: (i, 0)),
            pl.BlockSpec((BN, D_HID), lambda i: (i, 0)),
            pl.BlockSpec(w1m.shape, lambda i: (0, 0)),
            pl.BlockSpec(w1f.shape, lambda i: (0, 0)),
            pl.BlockSpec(w1h.shape, lambda i: (0, 0)),
            pl.BlockSpec(b1r.shape, lambda i: (0, 0)),
            pl.BlockSpec(W2.shape, lambda i: (0, 0)),
            pl.BlockSpec(b2r.shape, lambda i: (0, 0)),
            pl.BlockSpec(W3.shape, lambda i: (0, 0)),
            pl.BlockSpec(b3r.shape, lambda i: (0, 0)),
        ],
        out_specs=pl.BlockSpec((BN, OUT), lambda i: (i, 0)),
        out_shape=jax.ShapeDtypeStruct((N, OUT), jnp.float32),
        compiler_params=pltpu.CompilerParams(
            dimension_semantics=("parallel",),
        ),
    )(mailbox, node_features, node_hidden_rep,
      w1m, w1f, w1h, b1r, W2, b2r, W3, b3r)


# speedup vs baseline: 1.5052x; 1.0336x over previous
import jax
import jax.numpy as jnp
from jax.experimental import pallas as pl
from jax.experimental.pallas import tpu as pltpu

N = 10000
DEG = 32
D_MSG = 128
D_FEAT = 128
D_HID = 128
IN = D_MSG + D_FEAT + D_HID
H = 256
OUT = 128

BN = 400


def _fused_body(mb_ref, nf_ref, nh_ref, w1_ref, b1_ref,
                w2_ref, b2_ref, w3_ref, b3_ref, o_ref):
    msum = jnp.sum(mb_ref[...], axis=1)
    h = (jnp.dot(msum, w1_ref[0:D_MSG, :],
                 preferred_element_type=jnp.float32)
         + jnp.dot(nf_ref[...], w1_ref[D_MSG:D_MSG + D_FEAT, :],
                   preferred_element_type=jnp.float32)
         + jnp.dot(nh_ref[...], w1_ref[D_MSG + D_FEAT:IN, :],
                   preferred_element_type=jnp.float32)
         + b1_ref[...][None, :])
    h = jnp.maximum(h, 0.0)
    h = jnp.dot(h, w2_ref[...], preferred_element_type=jnp.float32) + b2_ref[...][None, :]
    h = jnp.maximum(h, 0.0)
    o_ref[...] = jnp.dot(h, w3_ref[...], preferred_element_type=jnp.float32) + b3_ref[...][None, :]


@jax.jit
def kernel(mailbox, node_features, node_hidden_rep, W1, b1, W2, b2, W3, b3):
    grid = (N // BN,)

    return pl.pallas_call(
        _fused_body,
        grid=grid,
        in_specs=[
            pl.BlockSpec((BN, DEG, D_MSG), lambda i: (i, 0, 0)),
            pl.BlockSpec((BN, D_FEAT), lambda i: (i, 0)),
            pl.BlockSpec((BN, D_HID), lambda i: (i, 0)),
            pl.BlockSpec(W1.shape, lambda i: (0, 0)),
            pl.BlockSpec(b1.shape, lambda i: (0,)),
            pl.BlockSpec(W2.shape, lambda i: (0, 0)),
            pl.BlockSpec(b2.shape, lambda i: (0,)),
            pl.BlockSpec(W3.shape, lambda i: (0, 0)),
            pl.BlockSpec(b3.shape, lambda i: (0,)),
        ],
        out_specs=pl.BlockSpec((BN, OUT), lambda i: (i, 0)),
        out_shape=jax.ShapeDtypeStruct((N, OUT), jnp.float32),
        compiler_params=pltpu.CompilerParams(
            dimension_semantics=("parallel",),
        ),
    )(mailbox, node_features, node_hidden_rep, W1, b1, W2, b2, W3, b3)
